# baseline (speedup 1.0000x reference)
import jax
import jax.numpy as jnp
from jax.experimental import pallas as pl
from jax.experimental.pallas import tpu as pltpu

N = 32768
D_OUT = 5
BLK = 32768
GRID = N // BLK


def _tc_body(wt_ref, b_ref, x_ref, out_ref):
    x0 = x_ref[0, :]
    x1 = x_ref[1, :]
    x2 = x_ref[2, :]
    m = x0 > x1
    zeros = jnp.zeros((BLK,), jnp.float32)
    for j in range(D_OUT):
        r = (wt_ref[0, j] + b_ref[j]) + x2 * wt_ref[2, j]
        out_ref[j, :] = jnp.where(m, r, 0.0)
    for j in range(D_OUT, 8):
        out_ref[j, :] = zeros


def kernel(s2, W10, b10):
    s2t = s2.T

    out8 = pl.pallas_call(
        _tc_body,
        grid=(GRID,),
        out_shape=jax.ShapeDtypeStruct((8, N), jnp.float32),
        in_specs=[
            pl.BlockSpec(memory_space=pltpu.SMEM),
            pl.BlockSpec(memory_space=pltpu.SMEM),
            pl.BlockSpec((3, BLK), lambda i: (0, i)),
        ],
        out_specs=pl.BlockSpec((8, BLK), lambda i: (0, i)),
    )(W10.T, b10, s2t)
    return out8.T[:, :D_OUT]


# speedup vs baseline: 1.1269x; 1.1269x over previous
import jax
import jax.numpy as jnp
from jax.experimental import pallas as pl
from jax.experimental.pallas import tpu as pltpu

N = 32768
D_OUT = 5
BLK = 16384
GRID = N // BLK


def _tc_body(wt_ref, b_ref, x_ref, out_ref):
    x0 = x_ref[0, :]
    x1 = x_ref[1, :]
    x2 = x_ref[2, :]
    m = x0 > x1
    for j in range(D_OUT):
        r = (wt_ref[0, j] + b_ref[j]) + x2 * wt_ref[2, j]
        out_ref[j, :] = jnp.where(m, r, 0.0)


def kernel(s2, W10, b10):
    s2t = s2.T

    out8 = pl.pallas_call(
        _tc_body,
        grid=(GRID,),
        out_shape=jax.ShapeDtypeStruct((D_OUT, N), jnp.float32),
        in_specs=[
            pl.BlockSpec(memory_space=pltpu.SMEM),
            pl.BlockSpec(memory_space=pltpu.SMEM),
            pl.BlockSpec((3, BLK), lambda i: (0, i)),
        ],
        out_specs=pl.BlockSpec((D_OUT, BLK), lambda i: (0, i)),
    )(W10.T, b10, s2t)
    return out8.T
